# baseline (speedup 1.0000x reference)
import functools

import jax
import jax.numpy as jnp
from jax.experimental import pallas as pl

_HIDDEN = 128
_OUT = 2
_EPS = 1e-5
_INV_H = 1.0 / _HIDDEN


def _mlp_block_kernel(x_ref, a1_ref, c1_ref, b1f_ref, a2_ref, c2_ref,
                      b2f_ref, out_ref):
    x = x_ref[...]
    mu = jnp.mean(x, axis=-1, keepdims=True)
    xc = x - mu
    var = jnp.mean(xc * xc, axis=-1, keepdims=True)
    rs = jax.lax.rsqrt(var + _EPS)

    p = jnp.dot(x, a1_ref[...], preferred_element_type=jnp.float32)
    h = jax.nn.sigmoid(rs * p - (rs * mu) * c1_ref[...] + b1f_ref[...])

    mu2 = jnp.mean(h, axis=-1, keepdims=True)
    hc = h - mu2
    var2 = jnp.mean(hc * hc, axis=-1, keepdims=True)
    rs2 = jax.lax.rsqrt(var2 + _EPS)

    q = jnp.dot(h, a2_ref[...], preferred_element_type=jnp.float32)
    out_ref[...] = rs2 * q - (rs2 * mu2) * c2_ref[...] + b2f_ref[...]


@functools.partial(jax.jit, static_argnames=("block_rows",))
def _run(causal, ln1_w, ln1_b, W1, b1, ln2_w, ln2_b, W2, b2, block_rows=4000):
    n_rows = causal.shape[0]
    grid = (n_rows // block_rows,)

    A1 = ln1_w[:, None] * W1.T
    b1f = b1 + ln1_b @ W1.T
    c1 = jnp.sum(A1, axis=0)
    A2 = ln2_w[:, None] * W2.T
    b2f = b2 + ln2_b @ W2.T
    c2 = jnp.sum(A2, axis=0)

    rep = lambda s: pl.BlockSpec(s, lambda i: (0, 0))
    out = pl.pallas_call(
        _mlp_block_kernel,
        grid=grid,
        in_specs=[
            pl.BlockSpec((block_rows, _HIDDEN), lambda i: (i, 0)),
            rep((_HIDDEN, _HIDDEN)),
            rep((1, _HIDDEN)),
            rep((1, _HIDDEN)),
            rep((_HIDDEN, _OUT)),
            rep((1, _OUT)),
            rep((1, _OUT)),
        ],
        out_specs=pl.BlockSpec((block_rows, _OUT), lambda i: (i, 0)),
        out_shape=jax.ShapeDtypeStruct((n_rows, _OUT), jnp.float32),
    )(
        causal,
        A1,
        c1.reshape(1, _HIDDEN),
        b1f.reshape(1, _HIDDEN),
        A2,
        c2.reshape(1, _OUT),
        b2f.reshape(1, _OUT),
    )
    return out


def kernel(causal, ln1_w, ln1_b, W1, b1, ln2_w, ln2_b, W2, b2):
    return _run(causal, ln1_w, ln1_b, W1, b1, ln2_w, ln2_b, W2, b2)


# speedup vs baseline: 1.0834x; 1.0834x over previous
import functools

import jax
import jax.numpy as jnp
from jax.experimental import pallas as pl

_HIDDEN = 128
_OUT = 2
_EPS = 1e-5
_INV_H = 1.0 / _HIDDEN


def _mlp_block_kernel(x_ref, a1_ref, b1f_ref, a2_ref, b2f_ref, out_ref):
    x = x_ref[...]
    mu = jnp.mean(x, axis=-1, keepdims=True)
    xc = x - mu
    var = jnp.mean(xc * xc, axis=-1, keepdims=True)
    xn = xc * jax.lax.rsqrt(var + _EPS)

    p = jnp.dot(xn, a1_ref[...], preferred_element_type=jnp.float32)
    h = jax.nn.sigmoid(p + b1f_ref[...])

    mu2 = jnp.mean(h, axis=-1, keepdims=True)
    hc = h - mu2
    var2 = jnp.mean(hc * hc, axis=-1, keepdims=True)
    hn = hc * jax.lax.rsqrt(var2 + _EPS)

    q = jnp.dot(hn, a2_ref[...], preferred_element_type=jnp.float32)
    out_ref[...] = q + b2f_ref[...]


@functools.partial(jax.jit, static_argnames=("block_rows",))
def _run(causal, ln1_w, ln1_b, W1, b1, ln2_w, ln2_b, W2, b2, block_rows=4000):
    n_rows = causal.shape[0]
    grid = (n_rows // block_rows,)

    A1 = ln1_w[:, None] * W1.T
    b1f = b1 + ln1_b @ W1.T
    A2 = ln2_w[:, None] * W2.T
    b2f = b2 + ln2_b @ W2.T

    rep = lambda s: pl.BlockSpec(s, lambda i: (0, 0))
    out = pl.pallas_call(
        _mlp_block_kernel,
        grid=grid,
        in_specs=[
            pl.BlockSpec((block_rows, _HIDDEN), lambda i: (i, 0)),
            rep((_HIDDEN, _HIDDEN)),
            rep((1, _HIDDEN)),
            rep((_HIDDEN, _OUT)),
            rep((1, _OUT)),
        ],
        out_specs=pl.BlockSpec((block_rows, _OUT), lambda i: (i, 0)),
        out_shape=jax.ShapeDtypeStruct((n_rows, _OUT), jnp.float32),
    )(
        causal,
        A1,
        b1f.reshape(1, _HIDDEN),
        A2,
        b2f.reshape(1, _OUT),
    )
    return out


def kernel(causal, ln1_w, ln1_b, W1, b1, ln2_w, ln2_b, W2, b2):
    return _run(causal, ln1_w, ln1_b, W1, b1, ln2_w, ln2_b, W2, b2)


# speedup vs baseline: 1.1632x; 1.0737x over previous
import functools

import jax
import jax.numpy as jnp
from jax.experimental import pallas as pl

_HIDDEN = 128
_OUT = 2
_EPS = 1e-5

_DN = (((1,), (1,)), ((), ()))


def _mlp_block_kernel(x_ref, ln1w_ref, ln1b_ref, w1_ref, b1_ref,
                      ln2w_ref, ln2b_ref, w2_ref, b2_ref, out_ref):
    x = x_ref[...]
    mu = jnp.mean(x, axis=-1, keepdims=True)
    xc = x - mu
    var = jnp.mean(xc * xc, axis=-1, keepdims=True)
    xn = xc * jax.lax.rsqrt(var + _EPS)
    xn = xn * ln1w_ref[...] + ln1b_ref[...]

    p = jax.lax.dot_general(xn, w1_ref[...], _DN,
                            preferred_element_type=jnp.float32)
    h = jax.nn.sigmoid(p + b1_ref[...])

    mu2 = jnp.mean(h, axis=-1, keepdims=True)
    hc = h - mu2
    var2 = jnp.mean(hc * hc, axis=-1, keepdims=True)
    hn = hc * jax.lax.rsqrt(var2 + _EPS)
    hn = hn * ln2w_ref[...] + ln2b_ref[...]

    q = jax.lax.dot_general(hn, w2_ref[...], _DN,
                            preferred_element_type=jnp.float32)
    out_ref[...] = q + b2_ref[...]


@functools.partial(jax.jit, static_argnames=("block_rows",))
def _run(causal, ln1_w, ln1_b, W1, b1, ln2_w, ln2_b, W2, b2, block_rows=4000):
    n_rows = causal.shape[0]
    grid = (n_rows // block_rows,)

    rep = lambda s: pl.BlockSpec(s, lambda i: (0, 0))
    out = pl.pallas_call(
        _mlp_block_kernel,
        grid=grid,
        in_specs=[
            pl.BlockSpec((block_rows, _HIDDEN), lambda i: (i, 0)),
            rep((1, _HIDDEN)),
            rep((1, _HIDDEN)),
            rep((_HIDDEN, _HIDDEN)),
            rep((1, _HIDDEN)),
            rep((1, _HIDDEN)),
            rep((1, _HIDDEN)),
            rep((_OUT, _HIDDEN)),
            rep((1, _OUT)),
        ],
        out_specs=pl.BlockSpec((block_rows, _OUT), lambda i: (i, 0)),
        out_shape=jax.ShapeDtypeStruct((n_rows, _OUT), jnp.float32),
    )(
        causal,
        ln1_w.reshape(1, _HIDDEN),
        ln1_b.reshape(1, _HIDDEN),
        W1,
        b1.reshape(1, _HIDDEN),
        ln2_w.reshape(1, _HIDDEN),
        ln2_b.reshape(1, _HIDDEN),
        W2,
        b2.reshape(1, _OUT),
    )
    return out


def kernel(causal, ln1_w, ln1_b, W1, b1, ln2_w, ln2_b, W2, b2):
    return _run(causal, ln1_w, ln1_b, W1, b1, ln2_w, ln2_b, W2, b2)
